# baseline (speedup 1.0000x reference)
import functools

import jax
import jax.numpy as jnp
from jax import lax
from jax.experimental import pallas as pl
from jax.experimental.pallas import tpu as pltpu
from jax.experimental.pallas import tpu_sc as plsc

VOCAB = 1000000
D_MODEL = 64
SEQ = 200
BATCH = 1024
N_FLAT = BATCH * SEQ


def _make_kernel():
    info = plsc.get_sparse_core_info()
    nc, ns = info.num_cores, info.num_subcores
    nw = nc * ns
    d_per_w = D_MODEL // nw
    assert D_MODEL % nw == 0

    mesh = plsc.VectorSubcoreMesh(core_axis_name="c", subcore_axis_name="s")

    @functools.partial(
        pl.kernel,
        mesh=mesh,
        out_type=jax.ShapeDtypeStruct((SEQ * D_MODEL * BATCH,), jnp.float32),
        scratch_types=[
            [pltpu.VMEM((BATCH,), jnp.int32) for _ in range(2)],
            [pltpu.VMEM((BATCH,), jnp.int32) for _ in range(4)],
            [pltpu.VMEM((BATCH,), jnp.float32) for _ in range(4)],
            pltpu.VMEM((SEQ * D_MODEL,), jnp.float32),
            [pltpu.SemaphoreType.DMA for _ in range(2)],
            [pltpu.SemaphoreType.DMA for _ in range(4)],
            [pltpu.SemaphoreType.DMA for _ in range(4)],
        ],
        compiler_params=pltpu.CompilerParams(
            use_tc_tiling_on_sc=False, needs_layout_passes=False
        ),
    )
    def emb_kernel(xt_hbm, tabf_hbm, posf_hbm, out_hbm, xrows, idxs, bufs,
                   pos_v, xsems, gsems, ssems):
        wid = lax.axis_index("s") * nc + lax.axis_index("c")
        dbase = wid * d_per_w * VOCAB
        pltpu.sync_copy(posf_hbm, pos_v)

        def x_start(s, slot):
            pltpu.async_copy(xt_hbm.at[pl.ds(s * BATCH, BATCH)], xrows[slot],
                             xsems[slot])

        def x_wait(slot):
            pltpu.make_async_copy(xt_hbm.at[pl.ds(0, BATCH)], xrows[slot],
                                  xsems[slot]).wait()

        def idx_compute(slot):
            xr, i0, i1 = xrows[slot], idxs[2 * slot], idxs[2 * slot + 1]

            def body(j, carry):
                sl = pl.ds(j * 16, 16)
                v = xr[sl]
                i0[sl] = v + dbase
                i1[sl] = v + (dbase + VOCAB)
                return carry

            lax.fori_loop(0, BATCH // 16, body, 0, unroll=8)

        def ga_fire(u, slot):
            k = 2 * slot + u
            for seg in range(BATCH // 128):
                sl = pl.ds(seg * 128, 128)
                pltpu.async_copy(tabf_hbm.at[idxs[k].at[sl]],
                                 bufs[k].at[sl], gsems[k])

        def ga_drain(u, slot):
            k = 2 * slot + u
            for seg in range(BATCH // 128):
                sl = pl.ds(seg * 128, 128)
                pltpu.make_async_copy(tabf_hbm.at[idxs[k].at[sl]],
                                     bufs[k].at[sl], gsems[k]).wait()

        def pos_add(s, u, slot):
            row = s * D_MODEL + wid * d_per_w + u
            splat_idx = jax.lax.iota(jnp.int32, 16) * 0 + row
            pvec = plsc.load_gather(pos_v, [splat_idx])
            buf = bufs[2 * slot + u]

            def body(j, carry):
                sl = pl.ds(j * 16, 16)
                buf[sl] = buf[sl] + pvec
                return carry

            lax.fori_loop(0, BATCH // 16, body, 0, unroll=8)

        def st_start(s, u, slot):
            row = s * D_MODEL + wid * d_per_w + u
            k = 2 * slot + u
            pltpu.async_copy(bufs[k], out_hbm.at[pl.ds(row * BATCH, BATCH)],
                             ssems[k])

        def st_wait(u, slot):
            k = 2 * slot + u
            pltpu.make_async_copy(bufs[k], out_hbm.at[pl.ds(0, BATCH)],
                                  ssems[k]).wait()

        x_start(0, 0)
        x_start(1, 1)

        def outer(c0, carry):
            for u_s in range(2):
                s = c0 * 2 + u_s
                slot, oslot = u_s, 1 - u_s
                x_wait(slot)
                idx_compute(slot)

                @pl.when(c0 <= (SEQ // 2) - 2)
                def _():
                    x_start(s + 2, slot)

                @pl.when(c0 >= 1)
                def _():
                    st_wait(0, slot)
                    st_wait(1, slot)

                ga_fire(0, slot)
                ga_fire(1, slot)

                if u_s == 1:
                    ga_drain(0, oslot)
                    pos_add(s - 1, 0, oslot)
                    st_start(s - 1, 0, oslot)
                    ga_drain(1, oslot)
                    pos_add(s - 1, 1, oslot)
                    st_start(s - 1, 1, oslot)
                else:
                    @pl.when(c0 >= 1)
                    def _():
                        ga_drain(0, oslot)
                        pos_add(s - 1, 0, oslot)
                        st_start(s - 1, 0, oslot)
                        ga_drain(1, oslot)
                        pos_add(s - 1, 1, oslot)
                        st_start(s - 1, 1, oslot)
            return carry

        lax.fori_loop(0, SEQ // 2, outer, 0, unroll=False)
        ga_drain(0, 1)
        pos_add(SEQ - 1, 0, 1)
        st_start(SEQ - 1, 0, 1)
        ga_drain(1, 1)
        pos_add(SEQ - 1, 1, 1)
        st_start(SEQ - 1, 1, 1)
        st_wait(0, 0)
        st_wait(1, 0)
        st_wait(0, 1)
        st_wait(1, 1)

    return emb_kernel


_emb_kernel = _make_kernel()


@jax.jit
def kernel(x, token_table, pos_embed):
    seq = x.shape[1]
    xt = x.T.astype(jnp.int32).reshape(-1)
    tabf = token_table.T.reshape(-1)
    pos2d = pos_embed[0, :seq, :].astype(jnp.float32)
    posf = pos2d.reshape(-1)
    out_flat = _emb_kernel(xt, tabf, posf)
    return out_flat.reshape(seq, D_MODEL, BATCH).transpose(2, 0, 1)


# speedup vs baseline: 6.2792x; 6.2792x over previous
import functools

import jax
import jax.numpy as jnp
from jax import lax
from jax.experimental import pallas as pl
from jax.experimental.pallas import tpu as pltpu
from jax.experimental.pallas import tpu_sc as plsc

VOCAB = 1000000
D_MODEL = 64
SEQ = 200
BATCH = 1024
N_FLAT = BATCH * SEQ

CHUNK = 128
NBUF = 5
PF_AHEAD = 4
GA_AHEAD = 3


def _make_kernel():
    info = plsc.get_sparse_core_info()
    nc, ns = info.num_cores, info.num_subcores
    nw = nc * ns
    per_w = N_FLAT // nw
    assert N_FLAT % nw == 0 and per_w % CHUNK == 0
    n_chunks = per_w // CHUNK
    assert n_chunks % NBUF == 0
    n_outer = n_chunks // NBUF

    mesh = plsc.VectorSubcoreMesh(core_axis_name="c", subcore_axis_name="s")

    @functools.partial(
        pl.kernel,
        mesh=mesh,
        out_type=jax.ShapeDtypeStruct((N_FLAT, D_MODEL), jnp.float32),
        scratch_types=[
            pltpu.VMEM((per_w,), jnp.int32),
            [pltpu.VMEM((CHUNK, D_MODEL), jnp.float32) for _ in range(NBUF)],
            [pltpu.SemaphoreType.DMA for _ in range(NBUF)],
            [pltpu.SemaphoreType.DMA for _ in range(NBUF)],
            [pltpu.SemaphoreType.DMA for _ in range(NBUF)],
        ],
        compiler_params=pltpu.CompilerParams(use_tc_tiling_on_sc=False),
    )
    def emb_kernel(x_hbm, tab_hbm, pos2_hbm, out_hbm, idx_v, bufs, psems,
                   gsems, ssems):
        wid = lax.axis_index("s") * nc + lax.axis_index("c")
        base = wid * per_w
        pltpu.sync_copy(x_hbm.at[pl.ds(base, per_w)], idx_v)

        def pf_start(c, b):
            t0 = lax.rem(c * CHUNK, SEQ)
            pltpu.async_copy(pos2_hbm.at[pl.ds(t0, CHUNK)], bufs[b], psems[b])

        def pf_wait(b):
            pltpu.make_async_copy(
                pos2_hbm.at[pl.ds(0, CHUNK)], bufs[b], psems[b]
            ).wait()

        def ga_start(c, b):
            off = pl.multiple_of(c * CHUNK, CHUNK)
            pltpu.async_copy(
                tab_hbm.at[idx_v.at[pl.ds(off, CHUNK)]], bufs[b], gsems[b],
                add=True,
            )

        def ga_wait(b):
            pltpu.make_async_copy(
                tab_hbm.at[idx_v.at[pl.ds(0, CHUNK)]], bufs[b], gsems[b]
            ).wait()

        def store_start(c, b):
            off = pl.multiple_of(c * CHUNK, CHUNK)
            pltpu.async_copy(bufs[b], out_hbm.at[pl.ds(base + off, CHUNK)],
                             ssems[b])

        def store_wait(b):
            pltpu.make_async_copy(
                bufs[b], out_hbm.at[pl.ds(base, CHUNK)], ssems[b]
            ).wait()

        for c in range(PF_AHEAD):
            pf_start(c, c)
        for c in range(GA_AHEAD):
            pf_wait(c)
            ga_start(c, c)

        def outer(c0, carry):
            for b in range(NBUF):
                c = c0 * NBUF + b

                ga_wait(b)
                store_start(c, b)

                bg = (b + GA_AHEAD) % NBUF
                if b < NBUF - GA_AHEAD:
                    pf_wait(bg)
                    ga_start(c + GA_AHEAD, bg)
                else:
                    @pl.when(c0 < n_outer - 1)
                    def _():
                        pf_wait(bg)
                        ga_start(c + GA_AHEAD, bg)

                bp = (b + PF_AHEAD) % NBUF
                if b >= 1:
                    store_wait(bp)
                else:
                    @pl.when(c0 >= 1)
                    def _():
                        store_wait(bp)

                if b < NBUF - PF_AHEAD:
                    pf_start(c + PF_AHEAD, bp)
                else:
                    @pl.when(c0 < n_outer - 1)
                    def _():
                        pf_start(c + PF_AHEAD, bp)
            return carry

        lax.fori_loop(0, n_outer, outer, 0, unroll=False)
        store_wait((n_chunks - 1) % NBUF)

    return emb_kernel


_emb_kernel = _make_kernel()


@jax.jit
def kernel(x, token_table, pos_embed):
    seq = x.shape[1]
    x_flat = x.reshape(-1).astype(jnp.int32)
    pos = pos_embed[0, :seq, :].astype(jnp.float32)
    pos2 = jnp.concatenate([pos, pos], axis=0)
    out_flat = _emb_kernel(x_flat, token_table, pos2)
    return out_flat.reshape(x.shape[0], seq, D_MODEL)
